# baseline (speedup 1.0000x reference)
import functools

import jax
import jax.numpy as jnp
from jax import lax
from jax.experimental import pallas as pl
from jax.experimental.pallas import tpu as pltpu
from jax.experimental.pallas import tpu_sc as plsc

_WIN = 32


def _sc_body(img_hbm, pos_hbm, w_hbm, out_hbm, pos_v, w_v, band_v):
    info = plsc.get_sparse_core_info()
    NC, NS, L = info.num_cores, info.num_subcores, info.num_lanes
    wid = lax.axis_index("s") * NC + lax.axis_index("c")
    n_img = img_hbm.shape[0] // (NC * NS)
    base = wid * n_img
    pltpu.sync_copy(pos_hbm.at[pl.ds(base * 2, n_img * 2)], pos_v)
    pltpu.sync_copy(w_hbm, w_v)
    pos_vec = pos_v[...]
    lanes = lax.iota(jnp.int32, L)
    for j in range(n_img):
        k = base + j
        p0 = pos_vec[2 * j]
        p1 = pos_vec[2 * j + 1]
        pltpu.sync_copy(img_hbm.at[k], out_hbm.at[k])
        pltpu.sync_copy(img_hbm.at[k, pl.ds(p0, _WIN), :], band_v)
        for r in range(_WIN):
            row_idx = jnp.full((L,), r, dtype=jnp.int32)
            for h in range(_WIN // L):
                col_idx = p1 + h * L + lanes
                plsc.store_scatter(
                    band_v, [row_idx, col_idx], w_v[r, pl.ds(h * L, L)]
                )
        pltpu.sync_copy(band_v, out_hbm.at[k, pl.ds(p0, _WIN), :])


def kernel(images, position, W):
    B, H, Wimg = images.shape
    info = plsc.get_sparse_core_info()
    n_img = B // (info.num_cores * info.num_subcores)
    mesh = plsc.VectorSubcoreMesh(core_axis_name="c", subcore_axis_name="s")
    f = functools.partial(
        pl.kernel,
        out_type=jax.ShapeDtypeStruct(images.shape, images.dtype),
        mesh=mesh,
        scratch_types=[
            pltpu.VMEM((n_img * 2,), jnp.int32),
            pltpu.VMEM((_WIN, _WIN), jnp.float32),
            pltpu.VMEM((_WIN, Wimg), jnp.float32),
        ],
        compiler_params=pltpu.CompilerParams(
            use_tc_tiling_on_sc=False, needs_layout_passes=False
        ),
    )(_sc_body)
    return f(images, position.astype(jnp.int32).reshape(-1), W)
